# baseline (speedup 1.0000x reference)
import functools

import jax
import jax.numpy as jnp
from jax import lax
from jax.experimental import pallas as pl
from jax.experimental.pallas import tpu as pltpu
from jax.experimental.pallas import tpu_sc as plsc

D = 64
B = 16384
NC, NS, L = 2, 16, 16
NW = NC * NS
BPW = B // NW
CHUNK = 128
NCHUNK = BPW // CHUNK


def _dot_body(emb, ni, nj, out, ni_v, nj_v, rows_i, rows_j, part_t, out_v, sem):
    wid = lax.axis_index("s") * NC + lax.axis_index("c")
    base = wid * BPW
    pltpu.sync_copy(ni.at[pl.ds(base, BPW)], ni_v)
    pltpu.sync_copy(nj.at[pl.ds(base, BPW)], nj_v)
    copies = []
    for c in range(NCHUNK):
        sl = pl.ds(c * CHUNK, CHUNK)
        copies.append(pltpu.async_copy(emb.at[ni_v.at[sl]], rows_i.at[sl], sem))
        copies.append(pltpu.async_copy(emb.at[nj_v.at[sl]], rows_j.at[sl], sem))
    for cp in copies:
        cp.wait()

    col_idx = lax.iota(jnp.int32, L) * (BPW + 1)

    def body(b, carry):
        p = rows_i[b, pl.ds(0, L)] * rows_j[b, pl.ds(0, L)]
        for c in range(1, D // L):
            p = p + rows_i[b, pl.ds(c * L, L)] * rows_j[b, pl.ds(c * L, L)]
        plsc.store_scatter(part_t, [col_idx + b], p)
        return carry

    lax.fori_loop(0, BPW, body, 0)

    def body2(g, carry):
        cb = g * L
        acc = part_t[pl.ds(cb, L)]
        for l in range(1, L):
            acc = acc + part_t[pl.ds(l * (BPW + 1) + cb, L)]
        out_v[pl.ds(cb, L)] = acc
        return carry

    lax.fori_loop(0, BPW // L, body2, 0)
    pltpu.sync_copy(out_v, out.at[pl.ds(base, BPW)])


@jax.jit
def _run(node_i, node_j, embeddings):
    mesh = plsc.VectorSubcoreMesh(core_axis_name="c", subcore_axis_name="s")
    f = functools.partial(
        pl.kernel,
        mesh=mesh,
        out_type=jax.ShapeDtypeStruct((B,), jnp.float32),
        compiler_params=pltpu.CompilerParams(
            needs_layout_passes=False, use_tc_tiling_on_sc=False
        ),
        scratch_types=[
            pltpu.VMEM((BPW,), jnp.int32),
            pltpu.VMEM((BPW,), jnp.int32),
            pltpu.VMEM((BPW, D), jnp.float32),
            pltpu.VMEM((BPW, D), jnp.float32),
            pltpu.VMEM((L * (BPW + 1),), jnp.float32),
            pltpu.VMEM((BPW,), jnp.float32),
            pltpu.SemaphoreType.DMA,
        ],
    )(_dot_body)
    return f(embeddings, node_i, node_j)


def kernel(node_i, node_j, embeddings):
    return _run(node_i, node_j, embeddings)


# speedup vs baseline: 1.5233x; 1.5233x over previous
import functools

import jax
import jax.numpy as jnp
from jax import lax
from jax.experimental import pallas as pl
from jax.experimental.pallas import tpu as pltpu
from jax.experimental.pallas import tpu_sc as plsc

D = 64
B = 16384
NC, NS, L = 2, 16, 16
NW = NC * NS
BPW = B // NW
KB = 16
NCH = BPW // KB
PSTRIDE = BPW + 1


def _dot_body(emb, ni, nj, out,
              ni_v, nj_v,
              bi0, bi1, bj0, bj1, part_t, out_v, sem0, sem1):
    wid = lax.axis_index("s") * NC + lax.axis_index("c")
    base = wid * BPW
    pltpu.sync_copy(ni.at[pl.ds(base, BPW)], ni_v.at[pl.ds(0, BPW)])
    pltpu.sync_copy(nj.at[pl.ds(base, BPW)], nj_v.at[pl.ds(0, BPW)])
    zeros = jnp.zeros((L,), jnp.int32)
    ni_v[pl.ds(BPW, KB)] = zeros
    nj_v[pl.ds(BPW, KB)] = zeros

    bufs_i = (bi0, bi1)
    bufs_j = (bj0, bj1)
    sems = (sem0, sem1)

    def fire(c, buf):
        vi = ni_v[pl.ds(c * KB, KB)]
        vj = nj_v[pl.ds(c * KB, KB)]
        for k in range(KB):
            pltpu.async_copy(emb.at[vi[k]], bufs_i[buf].at[k], sems[buf])
            pltpu.async_copy(emb.at[vj[k]], bufs_j[buf].at[k], sems[buf])

    def drain(buf):
        pltpu.make_async_copy(emb.at[pl.ds(0, KB)],
                              bufs_i[buf], sems[buf]).wait()
        pltpu.make_async_copy(emb.at[pl.ds(0, KB)],
                              bufs_j[buf], sems[buf]).wait()

    fire(0, 0)
    col_base = lax.iota(jnp.int32, L) * PSTRIDE

    def chunk_compute(c, buf):
        bi, bj = bufs_i[buf], bufs_j[buf]
        for k in range(KB):
            p = bi[k, pl.ds(0, L)] * bj[k, pl.ds(0, L)]
            for cc in range(1, D // L):
                sl = pl.ds(cc * L, L)
                p = p + bi[k, sl] * bj[k, sl]
            plsc.store_scatter(part_t, [col_base + (c * KB + k)], p)

    def body(h, carry):
        for phase in range(2):
            c = h * 2 + phase
            fire(c + 1, phase ^ 1)
            drain(phase)
            chunk_compute(c, phase)
        return carry

    lax.fori_loop(0, NCH // 2, body, 0)
    drain(0)

    def body2(g, carry):
        cb = g * L
        acc = part_t[pl.ds(cb, L)]
        for l in range(1, L):
            acc = acc + part_t[pl.ds(l * PSTRIDE + cb, L)]
        out_v[pl.ds(cb, L)] = acc
        return carry

    lax.fori_loop(0, BPW // L, body2, 0)
    pltpu.sync_copy(out_v, out.at[pl.ds(base, BPW)])


@jax.jit
def _run(node_i, node_j, embeddings):
    mesh = plsc.VectorSubcoreMesh(core_axis_name="c", subcore_axis_name="s")
    f = functools.partial(
        pl.kernel,
        mesh=mesh,
        out_type=jax.ShapeDtypeStruct((B,), jnp.float32),
        compiler_params=pltpu.CompilerParams(
            needs_layout_passes=False, use_tc_tiling_on_sc=True
        ),
        scratch_types=[
            pltpu.VMEM((BPW + KB,), jnp.int32),
            pltpu.VMEM((BPW + KB,), jnp.int32),
            pltpu.VMEM((KB, D), jnp.float32),
            pltpu.VMEM((KB, D), jnp.float32),
            pltpu.VMEM((KB, D), jnp.float32),
            pltpu.VMEM((KB, D), jnp.float32),
            pltpu.VMEM((L * PSTRIDE,), jnp.float32),
            pltpu.VMEM((BPW,), jnp.float32),
            pltpu.SemaphoreType.DMA,
            pltpu.SemaphoreType.DMA,
        ],
    )(_dot_body)
    return f(embeddings, node_i, node_j)


def kernel(node_i, node_j, embeddings):
    return _run(node_i, node_j, embeddings)


# speedup vs baseline: 1.9644x; 1.2896x over previous
import functools

import jax
import jax.numpy as jnp
from jax import lax
from jax.experimental import pallas as pl
from jax.experimental.pallas import tpu as pltpu
from jax.experimental.pallas import tpu_sc as plsc

D = 64
B = 16384
V = 1000000
NC, NS, L = 2, 16, 16
NW = NC * NS
BPW = B // NW
EPW = 2 * BPW
CAP = EPW
NBLK = V // 128
MAXBLK = NBLK // NW + 1
PSTRIDE = BPW + 1

_params = pltpu.CompilerParams(
    needs_layout_passes=False, use_tc_tiling_on_sc=True
)
_mesh = lambda: plsc.VectorSubcoreMesh(
    core_axis_name="c", subcore_axis_name="s"
)


def _wid():
    return lax.axis_index("s") * NC + lax.axis_index("c")


def _bin_body(ni, nj, bins, cnts, ni_v, nj_v, owner_v, pack_v, bin_v, cv,
              cnt_s, sem):
    wid = _wid()
    base = wid * BPW
    pltpu.sync_copy(ni.at[pl.ds(base, BPW)], ni_v)
    pltpu.sync_copy(nj.at[pl.ds(base, BPW)], nj_v)

    for side, src in ((0, ni_v), (1, nj_v)):
        for g in range(BPW // L):
            sl = pl.ds(side * BPW + g * L, L)
            r = src[pl.ds(g * L, L)]
            rr = lax.shift_right_logical(r, 7)
            owner_v[sl] = rr & 31
            key = lax.shift_left(lax.shift_right_logical(rr, 5), 7) | (r & 127)
            slot = base + g * L + lax.iota(jnp.int32, L)
            pack_v[sl] = (lax.shift_left(key, 15) | (side << 14) | slot)

    for o in range(NW):
        cnt_s[o] = 0
    mask0 = lax.iota(jnp.int32, L) == 0

    def e_body(g, carry):
        ov = owner_v[pl.ds(g * L, L)]
        pv = pack_v[pl.ds(g * L, L)]
        for k in range(L):
            o = ov[k]
            c = cnt_s[o]
            pos = o * CAP + c
            plsc.store_scatter(bin_v, [jnp.full((L,), pos, jnp.int32)],
                               jnp.full((L,), pv[k], jnp.int32), mask=mask0)
            cnt_s[o] = c + 1
        return carry

    lax.fori_loop(0, EPW // L, e_body, 0)

    for o in range(NW):
        plsc.store_scatter(cv, [jnp.full((L,), o, jnp.int32)],
                           jnp.full((L,), cnt_s[o], jnp.int32), mask=mask0)

    copies = []
    for o in range(NW):
        copies.append(pltpu.async_copy(
            bin_v.at[pl.ds(o * CAP, CAP)],
            bins.at[pl.ds((wid * NW + o) * CAP, CAP)], sem))
    copies.append(pltpu.async_copy(cv, cnts.at[pl.ds(wid * NW, NW)], sem))
    for cp in copies:
        cp.wait()


def _gather_body(emb_t, tail, bins, cnts, rows,
                 cm_v, bin_in, sorted_v, blk0, blk1, pblk, rowst,
                 cnt_v, offs_v, starts_v,
                 sem0, sem1, wsem, nw_s):
    wid = _wid()
    mask0 = lax.iota(jnp.int32, L) == 0
    nw_ = NW

    def sread(ref, i):
        return ref[pl.ds(i, L)][0]

    def swrite(ref, i, val):
        plsc.store_scatter(ref, [jnp.full((L,), i, jnp.int32)],
                           jnp.full((L,), val, jnp.int32), mask=mask0)
    pltpu.sync_copy(cnts, cm_v.at[pl.ds(0, NW * NW)])

    def in_body(s, carry):
        pltpu.sync_copy(bins.at[pl.ds((s * NW + wid) * CAP, CAP)],
                        bin_in.at[pl.ds(s * CAP, CAP)])
        return carry

    lax.fori_loop(0, NW, in_body, 0)

    zeros = jnp.zeros((L,), jnp.int32)
    for t in range(0, 272, L):
        cnt_v[pl.ds(t, L)] = zeros

    def count_src(s, carry):
        c_s = cm_v[pl.ds(s * NW + wid, L)][0]

        def g_body(g, carry2):
            v = bin_in[pl.ds(s * CAP + g * L, L)]
            for k in range(L):
                @pl.when(g * L + k < c_s)
                def _():
                    blk = lax.shift_right_logical(v[k], 22)
                    swrite(cnt_v, blk, sread(cnt_v, blk) + 1)
            return carry2

        lax.fori_loop(0, (c_s + L - 1) // L, g_body, 0)
        return carry

    lax.fori_loop(0, NW, count_src, 0)

    def p_body(t, run):
        swrite(offs_v, t, run)
        swrite(starts_v, t, run)
        return run + sread(cnt_v, t)

    run = lax.fori_loop(0, MAXBLK + 1, p_body, 0)
    swrite(starts_v, MAXBLK + 1, run)

    def place_src(s, carry):
        c_s = cm_v[pl.ds(s * NW + wid, L)][0]

        def g_body(g, carry2):
            v = bin_in[pl.ds(s * CAP + g * L, L)]
            for k in range(L):
                @pl.when(g * L + k < c_s)
                def _():
                    w = v[k]
                    blk = lax.shift_right_logical(w, 22)
                    p = offs_v[pl.ds(blk, L)][0]
                    plsc.store_scatter(
                        sorted_v, [jnp.full((L,), p, jnp.int32)],
                        jnp.full((L,), w, jnp.int32), mask=mask0)
                    swrite(offs_v, blk, p + 1)
            return carry2

        lax.fori_loop(0, (c_s + L - 1) // L, g_body, 0)
        return carry

    lax.fori_loop(0, NW, place_src, 0)

    nfull = (NBLK - 1 - wid) // NW + 1
    bufs = (blk0, blk1)
    sems = (sem0, sem1)
    c_idx = [lax.iota(jnp.int32, L) + q * L for q in range(4)]

    def fire(blk, buf):
        r0 = (blk * NW + wid) * 128
        pltpu.async_copy(emb_t.at[pl.ds(0, D), pl.ds(r0, 128)],
                         bufs[buf], sems[buf])

    def drain(buf):
        pltpu.make_async_copy(emb_t.at[pl.ds(0, D), pl.ds(0, 128)],
                              bufs[buf], sems[buf]).wait()

    def extract(bv, w, n):
        r1 = jnp.full((L,), lax.shift_right_logical(w, 15) & 127, jnp.int32)
        stg = n & (L - 1)
        for q in range(4):
            rowst[stg, pl.ds(q * L, L)] = plsc.load_gather(bv, [c_idx[q], r1])
        slot2 = (lax.shift_right_logical(w, 14) & 1) * B + (w & (B - 1))
        @pl.when(n >= L)
        def _():
            pltpu.make_async_copy(rows.at[0], rowst.at[0], wsem).wait()
        pltpu.async_copy(rowst.at[stg], rows.at[slot2], wsem)

    def process(bv, blk):
        start = sread(starts_v, blk)
        ne = sread(starts_v, blk + 1) - start

        def g_body(g, carry):
            v = sorted_v[pl.ds(start + g * L, L)]
            for k in range(L):
                @pl.when(g * L + k < ne)
                def _():
                    extract(bv, v[k], nw_s[0])
                    nw_s[0] = nw_s[0] + 1
            return carry

        lax.fori_loop(0, (ne + L - 1) // L, g_body, 0)

    nw_s[0] = 0
    fire(0, 0)

    def s_body2(h, carry):
        for phase in range(2):
            blk = h * 2 + phase
            @pl.when(blk < nfull)
            def _():
                nxt = jnp.minimum(blk + 1, nfull - 1)
                fire(nxt, phase ^ 1)
                drain(phase)
                process(bufs[phase], blk)
        return carry

    lax.fori_loop(0, (MAXBLK + 1) // 2, s_body2, 0)
    lastbuf = (nfull - 1) % 2
    @pl.when(lastbuf == 0)
    def _():
        drain(1)
    @pl.when(lastbuf == 1)
    def _():
        drain(0)

    @pl.when(wid == NBLK % NW)
    def _():
        pltpu.sync_copy(tail, pblk)
        tb = NBLK // NW
        start = sread(starts_v, tb)
        ne = sread(starts_v, tb + 1) - start

        def extract_tail(w, n):
            r1 = lax.shift_right_logical(w, 15) & 127
            stg = n & (L - 1)
            for q in range(4):
                rowst[stg, pl.ds(q * L, L)] = pblk[pl.ds(r1 * D + q * L, L)]
            slot2 = (lax.shift_right_logical(w, 14) & 1) * B + (w & (B - 1))
            @pl.when(n >= L)
            def _():
                pltpu.make_async_copy(rows.at[0], rowst.at[0], wsem).wait()
            pltpu.async_copy(rowst.at[stg], rows.at[slot2], wsem)

        def g_body(g, carry):
            v = sorted_v[pl.ds(start + g * L, L)]
            for k in range(L):
                @pl.when(g * L + k < ne)
                def _():
                    extract_tail(v[k], nw_s[0])
                    nw_s[0] = nw_s[0] + 1
            return carry

        lax.fori_loop(0, (ne + L - 1) // L, g_body, 0)

    nwr = nw_s[0]

    def d_body(i, carry):
        pltpu.make_async_copy(rows.at[0], rowst.at[0], wsem).wait()
        return carry

    lax.fori_loop(0, jnp.minimum(nwr, L), d_body, 0)


def _dot_body(rows, out, bi0, bi1, bj0, bj1, part_t, out_v, sem0, sem1):
    wid = _wid()
    base = wid * BPW
    KB = 128
    bufs_i = (bi0, bi1)
    bufs_j = (bj0, bj1)
    sems = (sem0, sem1)

    def fire(c, buf):
        sl0 = base + c * KB
        pltpu.async_copy(rows.at[pl.ds(sl0, KB)], bufs_i[buf], sems[buf])
        pltpu.async_copy(rows.at[pl.ds(B + sl0, KB)], bufs_j[buf], sems[buf])

    def drain(buf):
        pltpu.make_async_copy(rows.at[pl.ds(0, KB)],
                              bufs_i[buf], sems[buf]).wait()
        pltpu.make_async_copy(rows.at[pl.ds(0, KB)],
                              bufs_j[buf], sems[buf]).wait()

    fire(0, 0)
    col_base = lax.iota(jnp.int32, L) * PSTRIDE

    def chunk_compute(c, buf):
        bi, bj = bufs_i[buf], bufs_j[buf]

        def k_body(k, carry):
            p = bi[k, pl.ds(0, L)] * bj[k, pl.ds(0, L)]
            for cc in range(1, D // L):
                sl = pl.ds(cc * L, L)
                p = p + bi[k, sl] * bj[k, sl]
            plsc.store_scatter(part_t, [col_base + (c * KB + k)], p)
            return carry

        lax.fori_loop(0, KB, k_body, 0)

    for c in range(BPW // KB):
        if c + 1 < BPW // KB:
            fire(c + 1, (c + 1) & 1)
        drain(c & 1)
        chunk_compute(c, c & 1)

    def body2(g, carry):
        cb = g * L
        acc = part_t[pl.ds(cb, L)]
        for l in range(1, L):
            acc = acc + part_t[pl.ds(l * PSTRIDE + cb, L)]
        out_v[pl.ds(cb, L)] = acc
        return carry

    lax.fori_loop(0, BPW // L, body2, 0)
    pltpu.sync_copy(out_v, out.at[pl.ds(base, BPW)])


@jax.jit
def _run(node_i, node_j, embeddings):
    emb_t = embeddings.T

    bin_k = functools.partial(
        pl.kernel, mesh=_mesh(),
        out_type=(jax.ShapeDtypeStruct((NW * NW * CAP,), jnp.int32),
                  jax.ShapeDtypeStruct((NW * NW,), jnp.int32)),
        compiler_params=_params,
        scratch_types=[
            pltpu.VMEM((BPW,), jnp.int32),
            pltpu.VMEM((BPW,), jnp.int32),
            pltpu.VMEM((EPW,), jnp.int32),
            pltpu.VMEM((EPW,), jnp.int32),
            pltpu.VMEM((NW * CAP,), jnp.int32),
            pltpu.VMEM((NW,), jnp.int32),
            pltpu.SMEM((NW,), jnp.int32),
            pltpu.SemaphoreType.DMA,
        ],
    )(_bin_body)
    bins, cnts = bin_k(node_i, node_j)

    gather_k = functools.partial(
        pl.kernel, mesh=_mesh(),
        out_type=jax.ShapeDtypeStruct((2 * B, D), jnp.float32),
        compiler_params=_params,
        scratch_types=[
            pltpu.VMEM((NW * NW + L,), jnp.int32),
            pltpu.VMEM((NW * CAP,), jnp.int32),
            pltpu.VMEM((NW * CAP + L,), jnp.int32),
            pltpu.VMEM((D, 128), jnp.float32),
            pltpu.VMEM((D, 128), jnp.float32),
            pltpu.VMEM(((V - NBLK * 128) * D,), jnp.float32),
            pltpu.VMEM((L, D), jnp.float32),
            pltpu.VMEM((272,), jnp.int32),
            pltpu.VMEM((272,), jnp.int32),
            pltpu.VMEM((272,), jnp.int32),
            pltpu.SemaphoreType.DMA,
            pltpu.SemaphoreType.DMA,
            pltpu.SemaphoreType.DMA,
            pltpu.SMEM((1,), jnp.int32),
        ],
    )(_gather_body)
    tail = embeddings[NBLK * 128:].reshape(-1)
    rows = gather_k(emb_t, tail, bins, cnts)

    dot_k = functools.partial(
        pl.kernel, mesh=_mesh(),
        out_type=jax.ShapeDtypeStruct((B,), jnp.float32),
        compiler_params=_params,
        scratch_types=[
            pltpu.VMEM((128, D), jnp.float32),
            pltpu.VMEM((128, D), jnp.float32),
            pltpu.VMEM((128, D), jnp.float32),
            pltpu.VMEM((128, D), jnp.float32),
            pltpu.VMEM((L * PSTRIDE,), jnp.float32),
            pltpu.VMEM((BPW,), jnp.float32),
            pltpu.SemaphoreType.DMA,
            pltpu.SemaphoreType.DMA,
        ],
    )(_dot_body)
    return dot_k(rows)


def kernel(node_i, node_j, embeddings):
    return _run(node_i, node_j, embeddings)


# speedup vs baseline: 2.5344x; 1.2901x over previous
import functools

import jax
import jax.numpy as jnp
from jax import lax
from jax.experimental import pallas as pl
from jax.experimental.pallas import tpu as pltpu
from jax.experimental.pallas import tpu_sc as plsc

D = 64
B = 16384
V = 1000000
NC, NS, L = 2, 16, 16
NW = NC * NS
BPW = B // NW
EPW = 2 * BPW
CAP = EPW
W = 256
NBLK = V // W
MAXBLK = NBLK // NW + 1
PSTRIDE = BPW + 1

_params = pltpu.CompilerParams(
    needs_layout_passes=False, use_tc_tiling_on_sc=True
)
_mesh = lambda: plsc.VectorSubcoreMesh(
    core_axis_name="c", subcore_axis_name="s"
)


def _wid():
    return lax.axis_index("s") * NC + lax.axis_index("c")


def _bin_body(ni, nj, bins, cnts, ni_v, nj_v, owner_v, pack_v, bin_v, cv,
              cnt_s, sem):
    wid = _wid()
    base = wid * BPW
    pltpu.sync_copy(ni.at[pl.ds(base, BPW)], ni_v)
    pltpu.sync_copy(nj.at[pl.ds(base, BPW)], nj_v)

    for side, src in ((0, ni_v), (1, nj_v)):
        for g in range(BPW // L):
            sl = pl.ds(side * BPW + g * L, L)
            r = src[pl.ds(g * L, L)]
            rr = lax.shift_right_logical(r, 8)
            owner_v[sl] = rr & 31
            key = lax.shift_left(lax.shift_right_logical(rr, 5), 8) | (r & 255)
            slot = base + g * L + lax.iota(jnp.int32, L)
            pack_v[sl] = (lax.shift_left(key, 15) | (side << 14) | slot)

    for o in range(NW):
        cnt_s[o] = 0
    mask0 = lax.iota(jnp.int32, L) == 0

    def e_body(g, carry):
        ov = owner_v[pl.ds(g * L, L)]
        pv = pack_v[pl.ds(g * L, L)]
        for k in range(L):
            o = ov[k]
            c = cnt_s[o]
            pos = o * CAP + c
            plsc.store_scatter(bin_v, [jnp.full((L,), pos, jnp.int32)],
                               jnp.full((L,), pv[k], jnp.int32), mask=mask0)
            cnt_s[o] = c + 1
        return carry

    lax.fori_loop(0, EPW // L, e_body, 0)

    for o in range(NW):
        plsc.store_scatter(cv, [jnp.full((L,), o, jnp.int32)],
                           jnp.full((L,), cnt_s[o], jnp.int32), mask=mask0)

    copies = []
    for o in range(NW):
        copies.append(pltpu.async_copy(
            bin_v.at[pl.ds(o * CAP, CAP)],
            bins.at[pl.ds((wid * NW + o) * CAP, CAP)], sem))
    copies.append(pltpu.async_copy(cv, cnts.at[pl.ds(wid * NW, NW)], sem))
    for cp in copies:
        cp.wait()


def _gather_body(emb_t, tail, bins, cnts, rows,
                 cm_v, bin_in, sorted_v, blk0, blk1, blk2, pblk, rowst,
                 cnt_v, offs_v, starts_v,
                 sem0, sem1, sem2, wsem, nw_s):
    wid = _wid()
    mask0 = lax.iota(jnp.int32, L) == 0
    nw_ = NW

    def sread(ref, i):
        return ref[pl.ds(i, L)][0]

    def swrite(ref, i, val):
        plsc.store_scatter(ref, [jnp.full((L,), i, jnp.int32)],
                           jnp.full((L,), val, jnp.int32), mask=mask0)
    pltpu.sync_copy(cnts, cm_v.at[pl.ds(0, NW * NW)])

    def in_body(s, carry):
        pltpu.sync_copy(bins.at[pl.ds((s * NW + wid) * CAP, CAP)],
                        bin_in.at[pl.ds(s * CAP, CAP)])
        return carry

    lax.fori_loop(0, NW, in_body, 0)

    zeros = jnp.zeros((L,), jnp.int32)
    for t in range(0, 272, L):
        cnt_v[pl.ds(t, L)] = zeros

    def count_src(s, carry):
        c_s = cm_v[pl.ds(s * NW + wid, L)][0]

        def g_body(g, carry2):
            v = bin_in[pl.ds(s * CAP + g * L, L)]
            for k in range(L):
                @pl.when(g * L + k < c_s)
                def _():
                    blk = lax.shift_right_logical(v[k], 23)
                    swrite(cnt_v, blk, sread(cnt_v, blk) + 1)
            return carry2

        lax.fori_loop(0, (c_s + L - 1) // L, g_body, 0)
        return carry

    lax.fori_loop(0, NW, count_src, 0)

    def p_body(t, run):
        swrite(offs_v, t, run)
        swrite(starts_v, t, run)
        return run + sread(cnt_v, t)

    run = lax.fori_loop(0, MAXBLK + 1, p_body, 0)
    swrite(starts_v, MAXBLK + 1, run)

    def place_src(s, carry):
        c_s = cm_v[pl.ds(s * NW + wid, L)][0]

        def g_body(g, carry2):
            v = bin_in[pl.ds(s * CAP + g * L, L)]
            for k in range(L):
                @pl.when(g * L + k < c_s)
                def _():
                    w = v[k]
                    blk = lax.shift_right_logical(w, 23)
                    p = offs_v[pl.ds(blk, L)][0]
                    plsc.store_scatter(
                        sorted_v, [jnp.full((L,), p, jnp.int32)],
                        jnp.full((L,), w, jnp.int32), mask=mask0)
                    swrite(offs_v, blk, p + 1)
            return carry2

        lax.fori_loop(0, (c_s + L - 1) // L, g_body, 0)
        return carry

    lax.fori_loop(0, NW, place_src, 0)

    nfull = (NBLK - 1 - wid) // NW + 1
    bufs = (blk0, blk1, blk2)
    sems = (sem0, sem1, sem2)
    c_idx = [lax.iota(jnp.int32, L) + q * L for q in range(4)]

    def fire(blk, buf):
        r0 = (blk * NW + wid) * W
        pltpu.async_copy(emb_t.at[pl.ds(0, D), pl.ds(r0, W)],
                         bufs[buf], sems[buf])

    def drain(buf):
        pltpu.make_async_copy(emb_t.at[pl.ds(0, D), pl.ds(0, W)],
                              bufs[buf], sems[buf]).wait()

    def extract(bv, w, n):
        r1 = jnp.full((L,), lax.shift_right_logical(w, 15) & 255, jnp.int32)
        stg = n & (L - 1)
        for q in range(4):
            rowst[stg, pl.ds(q * L, L)] = plsc.load_gather(bv, [c_idx[q], r1])
        slot2 = (lax.shift_right_logical(w, 14) & 1) * B + (w & (B - 1))
        @pl.when(n >= L)
        def _():
            pltpu.make_async_copy(rows.at[0], rowst.at[0], wsem).wait()
        pltpu.async_copy(rowst.at[stg], rows.at[slot2], wsem)

    def process(bv, blk):
        start = sread(starts_v, blk)
        ne = sread(starts_v, blk + 1) - start

        def g_body(g, carry):
            v = sorted_v[pl.ds(start + g * L, L)]
            for k in range(L):
                @pl.when(g * L + k < ne)
                def _():
                    extract(bv, v[k], nw_s[0])
                    nw_s[0] = nw_s[0] + 1
            return carry

        lax.fori_loop(0, (ne + L - 1) // L, g_body, 0)

    nw_s[0] = 0
    fire(0, 0)
    fire(1, 1)

    def s_body3(h, carry):
        for phase in range(3):
            blk = h * 3 + phase
            @pl.when(blk < nfull)
            def _():
                nxt = jnp.minimum(blk + 2, nfull - 1)
                fire(nxt, (phase + 2) % 3)
                drain(phase % 3)
                process(bufs[phase % 3], blk)
        return carry

    lax.fori_loop(0, (MAXBLK + 3) // 3, s_body3, 0)
    lastbuf = (nfull - 1) % 3
    for q in range(3):
        @pl.when(lastbuf != q)
        def _(q=q):
            drain(q)

    @pl.when(wid == NBLK % NW)
    def _():
        pltpu.sync_copy(tail, pblk)
        tb = NBLK // NW
        start = sread(starts_v, tb)
        ne = sread(starts_v, tb + 1) - start

        def extract_tail(w, n):
            r1 = lax.shift_right_logical(w, 15) & 255
            stg = n & (L - 1)
            for q in range(4):
                rowst[stg, pl.ds(q * L, L)] = pblk[pl.ds(r1 * D + q * L, L)]
            slot2 = (lax.shift_right_logical(w, 14) & 1) * B + (w & (B - 1))
            @pl.when(n >= L)
            def _():
                pltpu.make_async_copy(rows.at[0], rowst.at[0], wsem).wait()
            pltpu.async_copy(rowst.at[stg], rows.at[slot2], wsem)

        def g_body(g, carry):
            v = sorted_v[pl.ds(start + g * L, L)]
            for k in range(L):
                @pl.when(g * L + k < ne)
                def _():
                    extract_tail(v[k], nw_s[0])
                    nw_s[0] = nw_s[0] + 1
            return carry

        lax.fori_loop(0, (ne + L - 1) // L, g_body, 0)

    nwr = nw_s[0]

    def d_body(i, carry):
        pltpu.make_async_copy(rows.at[0], rowst.at[0], wsem).wait()
        return carry

    lax.fori_loop(0, jnp.minimum(nwr, L), d_body, 0)


def _dot_body(rows, out, bi0, bi1, bj0, bj1, part_t, out_v, sem0, sem1):
    wid = _wid()
    base = wid * BPW
    KB = 128
    bufs_i = (bi0, bi1)
    bufs_j = (bj0, bj1)
    sems = (sem0, sem1)

    def fire(c, buf):
        sl0 = base + c * KB
        pltpu.async_copy(rows.at[pl.ds(sl0, KB)], bufs_i[buf], sems[buf])
        pltpu.async_copy(rows.at[pl.ds(B + sl0, KB)], bufs_j[buf], sems[buf])

    def drain(buf):
        pltpu.make_async_copy(rows.at[pl.ds(0, KB)],
                              bufs_i[buf], sems[buf]).wait()
        pltpu.make_async_copy(rows.at[pl.ds(0, KB)],
                              bufs_j[buf], sems[buf]).wait()

    fire(0, 0)
    col_base = lax.iota(jnp.int32, L) * PSTRIDE

    def chunk_compute(c, buf):
        bi, bj = bufs_i[buf], bufs_j[buf]

        def k_body(k, carry):
            p = bi[k, pl.ds(0, L)] * bj[k, pl.ds(0, L)]
            for cc in range(1, D // L):
                sl = pl.ds(cc * L, L)
                p = p + bi[k, sl] * bj[k, sl]
            plsc.store_scatter(part_t, [col_base + (c * KB + k)], p)
            return carry

        lax.fori_loop(0, KB, k_body, 0)

    for c in range(BPW // KB):
        if c + 1 < BPW // KB:
            fire(c + 1, (c + 1) & 1)
        drain(c & 1)
        chunk_compute(c, c & 1)

    def body2(g, carry):
        cb = g * L
        acc = part_t[pl.ds(cb, L)]
        for l in range(1, L):
            acc = acc + part_t[pl.ds(l * PSTRIDE + cb, L)]
        out_v[pl.ds(cb, L)] = acc
        return carry

    lax.fori_loop(0, BPW // L, body2, 0)
    pltpu.sync_copy(out_v, out.at[pl.ds(base, BPW)])


@jax.jit
def _run(node_i, node_j, embeddings):
    emb_t = embeddings.T

    bin_k = functools.partial(
        pl.kernel, mesh=_mesh(),
        out_type=(jax.ShapeDtypeStruct((NW * NW * CAP,), jnp.int32),
                  jax.ShapeDtypeStruct((NW * NW,), jnp.int32)),
        compiler_params=_params,
        scratch_types=[
            pltpu.VMEM((BPW,), jnp.int32),
            pltpu.VMEM((BPW,), jnp.int32),
            pltpu.VMEM((EPW,), jnp.int32),
            pltpu.VMEM((EPW,), jnp.int32),
            pltpu.VMEM((NW * CAP,), jnp.int32),
            pltpu.VMEM((NW,), jnp.int32),
            pltpu.SMEM((NW,), jnp.int32),
            pltpu.SemaphoreType.DMA,
        ],
    )(_bin_body)
    bins, cnts = bin_k(node_i, node_j)

    gather_k = functools.partial(
        pl.kernel, mesh=_mesh(),
        out_type=jax.ShapeDtypeStruct((2 * B, D), jnp.float32),
        compiler_params=_params,
        scratch_types=[
            pltpu.VMEM((NW * NW + L,), jnp.int32),
            pltpu.VMEM((NW * CAP,), jnp.int32),
            pltpu.VMEM((NW * CAP + L,), jnp.int32),
            pltpu.VMEM((D, W), jnp.float32),
            pltpu.VMEM((D, W), jnp.float32),
            pltpu.VMEM((D, W), jnp.float32),
            pltpu.VMEM(((V - NBLK * W) * D,), jnp.float32),
            pltpu.VMEM((L, D), jnp.float32),
            pltpu.VMEM((272,), jnp.int32),
            pltpu.VMEM((272,), jnp.int32),
            pltpu.VMEM((272,), jnp.int32),
            pltpu.SemaphoreType.DMA,
            pltpu.SemaphoreType.DMA,
            pltpu.SemaphoreType.DMA,
            pltpu.SemaphoreType.DMA,
            pltpu.SMEM((1,), jnp.int32),
        ],
    )(_gather_body)
    tail = embeddings[NBLK * W:].reshape(-1)
    rows = gather_k(emb_t, tail, bins, cnts)

    dot_k = functools.partial(
        pl.kernel, mesh=_mesh(),
        out_type=jax.ShapeDtypeStruct((B,), jnp.float32),
        compiler_params=_params,
        scratch_types=[
            pltpu.VMEM((128, D), jnp.float32),
            pltpu.VMEM((128, D), jnp.float32),
            pltpu.VMEM((128, D), jnp.float32),
            pltpu.VMEM((128, D), jnp.float32),
            pltpu.VMEM((L * PSTRIDE,), jnp.float32),
            pltpu.VMEM((BPW,), jnp.float32),
            pltpu.SemaphoreType.DMA,
            pltpu.SemaphoreType.DMA,
        ],
    )(_dot_body)
    return dot_k(rows)


def kernel(node_i, node_j, embeddings):
    return _run(node_i, node_j, embeddings)


# speedup vs baseline: 2.7297x; 1.0771x over previous
import functools

import jax
import jax.numpy as jnp
from jax import lax
from jax.experimental import pallas as pl
from jax.experimental.pallas import tpu as pltpu
from jax.experimental.pallas import tpu_sc as plsc

D = 64
B = 16384
V = 1000000
NC, NS, L = 2, 16, 16
NW = NC * NS
BPW = B // NW
EPW = 2 * BPW
CAP = EPW
W = 256
NBLK = V // W
MAXBLK = NBLK // NW + 1
PSTRIDE = BPW + 1

_params = pltpu.CompilerParams(
    needs_layout_passes=False, use_tc_tiling_on_sc=True
)
_mesh = lambda: plsc.VectorSubcoreMesh(
    core_axis_name="c", subcore_axis_name="s"
)


def _wid():
    return lax.axis_index("s") * NC + lax.axis_index("c")


def _bin_body(ni, nj, bins, cnts, ni_v, nj_v, owner_v, pack_v, bin_v, cv,
              cnt_s, sem):
    wid = _wid()
    base = wid * BPW
    pltpu.sync_copy(ni.at[pl.ds(base, BPW)], ni_v)
    pltpu.sync_copy(nj.at[pl.ds(base, BPW)], nj_v)

    for side, src in ((0, ni_v), (1, nj_v)):
        for g in range(BPW // L):
            sl = pl.ds(side * BPW + g * L, L)
            r = src[pl.ds(g * L, L)]
            rr = lax.shift_right_logical(r, 8)
            owner_v[sl] = rr & 31
            key = lax.shift_left(lax.shift_right_logical(rr, 5), 8) | (r & 255)
            slot = base + g * L + lax.iota(jnp.int32, L)
            pack_v[sl] = (lax.shift_left(key, 15) | (side << 14) | slot)

    for o in range(NW):
        cnt_s[o] = 0
    mask0 = lax.iota(jnp.int32, L) == 0

    def e_body(g, carry):
        ov = owner_v[pl.ds(g * L, L)]
        pv = pack_v[pl.ds(g * L, L)]
        for k in range(L):
            o = ov[k]
            c = cnt_s[o]
            pos = o * CAP + c
            plsc.store_scatter(bin_v, [jnp.full((L,), pos, jnp.int32)],
                               jnp.full((L,), pv[k], jnp.int32), mask=mask0)
            cnt_s[o] = c + 1
        return carry

    lax.fori_loop(0, EPW // L, e_body, 0)

    for o in range(NW):
        plsc.store_scatter(cv, [jnp.full((L,), o, jnp.int32)],
                           jnp.full((L,), cnt_s[o], jnp.int32), mask=mask0)

    copies = []
    for o in range(NW):
        copies.append(pltpu.async_copy(
            bin_v.at[pl.ds(o * CAP, CAP)],
            bins.at[pl.ds((o * NW + wid) * CAP, CAP)], sem))
    copies.append(pltpu.async_copy(cv, cnts.at[pl.ds(wid * NW, NW)], sem))
    for cp in copies:
        cp.wait()


def _gather_body(emb_t, tail, bins, cnts, rows,
                 cm_v, bin_in, sorted_v, blk0, blk1, blk2, pblk, rowst,
                 cnt_v, offs_v, starts_v,
                 sem0, sem1, sem2, wsem, nw_s):
    wid = _wid()
    mask0 = lax.iota(jnp.int32, L) == 0
    nw_ = NW

    def sread(ref, i):
        return ref[pl.ds(i, L)][0]

    def swrite(ref, i, val):
        plsc.store_scatter(ref, [jnp.full((L,), i, jnp.int32)],
                           jnp.full((L,), val, jnp.int32), mask=mask0)
    pltpu.sync_copy(cnts, cm_v.at[pl.ds(0, NW * NW)])
    pltpu.sync_copy(bins.at[pl.ds(wid * (NW * CAP), NW * CAP)], bin_in)

    nfull = (NBLK - 1 - wid) // NW + 1
    bufs = (blk0, blk1, blk2)
    sems = (sem0, sem1, sem2)

    def fire(blk, buf):
        r0 = (blk * NW + wid) * W
        pltpu.async_copy(emb_t.at[pl.ds(0, D), pl.ds(r0, W)],
                         bufs[buf], sems[buf])

    def drain(buf):
        pltpu.make_async_copy(emb_t.at[pl.ds(0, D), pl.ds(0, W)],
                              bufs[buf], sems[buf]).wait()

    fire(0, 0)
    fire(1, 1)

    zeros = jnp.zeros((L,), jnp.int32)
    for t in range(0, 272, L):
        cnt_v[pl.ds(t, L)] = zeros

    def count_src(s, carry):
        c_s = cm_v[pl.ds(s * NW + wid, L)][0]

        def g_body(g, carry2):
            v = bin_in[pl.ds(s * CAP + g * L, L)]
            for k in range(L):
                @pl.when(g * L + k < c_s)
                def _():
                    blk = lax.shift_right_logical(v[k], 23)
                    swrite(cnt_v, blk, sread(cnt_v, blk) + 1)
            return carry2

        lax.fori_loop(0, (c_s + L - 1) // L, g_body, 0)
        return carry

    lax.fori_loop(0, NW, count_src, 0)

    def p_body(t, run):
        swrite(offs_v, t, run)
        swrite(starts_v, t, run)
        return run + sread(cnt_v, t)

    run = lax.fori_loop(0, MAXBLK + 1, p_body, 0)
    swrite(starts_v, MAXBLK + 1, run)

    def place_src(s, carry):
        c_s = cm_v[pl.ds(s * NW + wid, L)][0]

        def g_body(g, carry2):
            v = bin_in[pl.ds(s * CAP + g * L, L)]
            for k in range(L):
                @pl.when(g * L + k < c_s)
                def _():
                    w = v[k]
                    blk = lax.shift_right_logical(w, 23)
                    p = offs_v[pl.ds(blk, L)][0]
                    plsc.store_scatter(
                        sorted_v, [jnp.full((L,), p, jnp.int32)],
                        jnp.full((L,), w, jnp.int32), mask=mask0)
                    swrite(offs_v, blk, p + 1)
            return carry2

        lax.fori_loop(0, (c_s + L - 1) // L, g_body, 0)
        return carry

    lax.fori_loop(0, NW, place_src, 0)

    c_idx = [lax.iota(jnp.int32, L) + q * L for q in range(4)]

    def extract(bv, w, n):
        r1 = jnp.full((L,), lax.shift_right_logical(w, 15) & 255, jnp.int32)
        stg = n & (L - 1)
        for q in range(4):
            rowst[stg, pl.ds(q * L, L)] = plsc.load_gather(bv, [c_idx[q], r1])
        slot2 = (lax.shift_right_logical(w, 14) & 1) * B + (w & (B - 1))
        @pl.when(n >= L)
        def _():
            pltpu.make_async_copy(rows.at[0], rowst.at[0], wsem).wait()
        pltpu.async_copy(rowst.at[stg], rows.at[slot2], wsem)

    def process(bv, blk):
        start = sread(starts_v, blk)
        ne = sread(starts_v, blk + 1) - start

        def g_body(g, carry):
            v = sorted_v[pl.ds(start + g * L, L)]
            for k in range(L):
                @pl.when(g * L + k < ne)
                def _():
                    extract(bv, v[k], nw_s[0])
                    nw_s[0] = nw_s[0] + 1
            return carry

        lax.fori_loop(0, (ne + L - 1) // L, g_body, 0)

    nw_s[0] = 0

    def s_body3(h, carry):
        for phase in range(3):
            blk = h * 3 + phase
            @pl.when(blk < nfull)
            def _():
                nxt = jnp.minimum(blk + 2, nfull - 1)
                fire(nxt, (phase + 2) % 3)
                drain(phase % 3)
                process(bufs[phase % 3], blk)
        return carry

    lax.fori_loop(0, (MAXBLK + 3) // 3, s_body3, 0)
    lastbuf = (nfull - 1) % 3
    for q in range(3):
        @pl.when(lastbuf != q)
        def _(q=q):
            drain(q)

    @pl.when(wid == NBLK % NW)
    def _():
        pltpu.sync_copy(tail, pblk)
        tb = NBLK // NW
        start = sread(starts_v, tb)
        ne = sread(starts_v, tb + 1) - start

        def extract_tail(w, n):
            r1 = lax.shift_right_logical(w, 15) & 255
            stg = n & (L - 1)
            for q in range(4):
                rowst[stg, pl.ds(q * L, L)] = pblk[pl.ds(r1 * D + q * L, L)]
            slot2 = (lax.shift_right_logical(w, 14) & 1) * B + (w & (B - 1))
            @pl.when(n >= L)
            def _():
                pltpu.make_async_copy(rows.at[0], rowst.at[0], wsem).wait()
            pltpu.async_copy(rowst.at[stg], rows.at[slot2], wsem)

        def g_body(g, carry):
            v = sorted_v[pl.ds(start + g * L, L)]
            for k in range(L):
                @pl.when(g * L + k < ne)
                def _():
                    extract_tail(v[k], nw_s[0])
                    nw_s[0] = nw_s[0] + 1
            return carry

        lax.fori_loop(0, (ne + L - 1) // L, g_body, 0)

    nwr = nw_s[0]

    def d_body(i, carry):
        pltpu.make_async_copy(rows.at[0], rowst.at[0], wsem).wait()
        return carry

    lax.fori_loop(0, jnp.minimum(nwr, L), d_body, 0)


def _dot_body(rows, out, bi0, bi1, bj0, bj1, part_t, out_v, sem0, sem1):
    wid = _wid()
    base = wid * BPW
    KB = 128
    bufs_i = (bi0, bi1)
    bufs_j = (bj0, bj1)
    sems = (sem0, sem1)

    def fire(c, buf):
        sl0 = base + c * KB
        pltpu.async_copy(rows.at[pl.ds(sl0, KB)], bufs_i[buf], sems[buf])
        pltpu.async_copy(rows.at[pl.ds(B + sl0, KB)], bufs_j[buf], sems[buf])

    def drain(buf):
        pltpu.make_async_copy(rows.at[pl.ds(0, KB)],
                              bufs_i[buf], sems[buf]).wait()
        pltpu.make_async_copy(rows.at[pl.ds(0, KB)],
                              bufs_j[buf], sems[buf]).wait()

    fire(0, 0)
    col_base = lax.iota(jnp.int32, L) * PSTRIDE

    def chunk_compute(c, buf):
        bi, bj = bufs_i[buf], bufs_j[buf]

        def k_body(k, carry):
            p = bi[k, pl.ds(0, L)] * bj[k, pl.ds(0, L)]
            for cc in range(1, D // L):
                sl = pl.ds(cc * L, L)
                p = p + bi[k, sl] * bj[k, sl]
            plsc.store_scatter(part_t, [col_base + (c * KB + k)], p)
            return carry

        lax.fori_loop(0, KB, k_body, 0)

    for c in range(BPW // KB):
        if c + 1 < BPW // KB:
            fire(c + 1, (c + 1) & 1)
        drain(c & 1)
        chunk_compute(c, c & 1)

    def body2(g, carry):
        cb = g * L
        acc = part_t[pl.ds(cb, L)]
        for l in range(1, L):
            acc = acc + part_t[pl.ds(l * PSTRIDE + cb, L)]
        out_v[pl.ds(cb, L)] = acc
        return carry

    lax.fori_loop(0, BPW // L, body2, 0)
    pltpu.sync_copy(out_v, out.at[pl.ds(base, BPW)])


@jax.jit
def _run(node_i, node_j, embeddings):
    emb_t = embeddings.T

    bin_k = functools.partial(
        pl.kernel, mesh=_mesh(),
        out_type=(jax.ShapeDtypeStruct((NW * NW * CAP,), jnp.int32),
                  jax.ShapeDtypeStruct((NW * NW,), jnp.int32)),
        compiler_params=_params,
        scratch_types=[
            pltpu.VMEM((BPW,), jnp.int32),
            pltpu.VMEM((BPW,), jnp.int32),
            pltpu.VMEM((EPW,), jnp.int32),
            pltpu.VMEM((EPW,), jnp.int32),
            pltpu.VMEM((NW * CAP,), jnp.int32),
            pltpu.VMEM((NW,), jnp.int32),
            pltpu.SMEM((NW,), jnp.int32),
            pltpu.SemaphoreType.DMA,
        ],
    )(_bin_body)
    bins, cnts = bin_k(node_i, node_j)

    gather_k = functools.partial(
        pl.kernel, mesh=_mesh(),
        out_type=jax.ShapeDtypeStruct((2 * B, D), jnp.float32),
        compiler_params=_params,
        scratch_types=[
            pltpu.VMEM((NW * NW + L,), jnp.int32),
            pltpu.VMEM((NW * CAP,), jnp.int32),
            pltpu.VMEM((NW * CAP + L,), jnp.int32),
            pltpu.VMEM((D, W), jnp.float32),
            pltpu.VMEM((D, W), jnp.float32),
            pltpu.VMEM((D, W), jnp.float32),
            pltpu.VMEM(((V - NBLK * W) * D,), jnp.float32),
            pltpu.VMEM((L, D), jnp.float32),
            pltpu.VMEM((272,), jnp.int32),
            pltpu.VMEM((272,), jnp.int32),
            pltpu.VMEM((272,), jnp.int32),
            pltpu.SemaphoreType.DMA,
            pltpu.SemaphoreType.DMA,
            pltpu.SemaphoreType.DMA,
            pltpu.SemaphoreType.DMA,
            pltpu.SMEM((1,), jnp.int32),
        ],
    )(_gather_body)
    tail = embeddings[NBLK * W:].reshape(-1)
    rows = gather_k(emb_t, tail, bins, cnts)

    dot_k = functools.partial(
        pl.kernel, mesh=_mesh(),
        out_type=jax.ShapeDtypeStruct((B,), jnp.float32),
        compiler_params=_params,
        scratch_types=[
            pltpu.VMEM((128, D), jnp.float32),
            pltpu.VMEM((128, D), jnp.float32),
            pltpu.VMEM((128, D), jnp.float32),
            pltpu.VMEM((128, D), jnp.float32),
            pltpu.VMEM((L * PSTRIDE,), jnp.float32),
            pltpu.VMEM((BPW,), jnp.float32),
            pltpu.SemaphoreType.DMA,
            pltpu.SemaphoreType.DMA,
        ],
    )(_dot_body)
    return dot_k(rows)


def kernel(node_i, node_j, embeddings):
    return _run(node_i, node_j, embeddings)
